# Initial kernel scaffold; baseline (speedup 1.0000x reference)
import jax
import jax.numpy as jnp
from jax.experimental import pallas as pl


def kernel(h, e, edge_index, W_emb_h, b_emb_h, W_emb_e, b_emb_e, W_conv, b_conv, gamma, beta, W_mlp0, b_mlp0, W_mlp1, b_mlp1, W_mlp2, b_mlp2):
    raise NotImplementedError("write your pallas kernel here")


# speedup vs baseline: 9.7764x; 9.7764x over previous
import functools

import jax
import jax.numpy as jnp
from jax import lax
from jax.experimental import pallas as pl
from jax.experimental.pallas import tpu as pltpu
from jax.experimental.pallas import tpu_sc as plsc

N = 10000
E = 320000
HID = 128

NC = 2
NS = 16
C = 128
NB = 2
NPAD = 10240

EPS = E // NS
NM2 = EPS // C
REM2 = EPS - NM2 * C
NR2 = NM2 // NB
HALFN = NPAD // 2
JR = 512
ACCR = HALFN + JR
RPS2 = HALFN // NS


@functools.cache
def _sc_kernels():
    sc_mesh = plsc.VectorSubcoreMesh(
        core_axis_name="c", subcore_axis_name="s", num_cores=NC,
        num_subcores=NS,
    )
    deg = functools.partial(
        pl.kernel,
        mesh=sc_mesh,
        out_type=jax.ShapeDtypeStruct((NC, 2, HALFN, HID), jnp.float32),
        scratch_types=[
            pltpu.VMEM((NM2, C), jnp.int32),
            pltpu.VMEM((REM2,), jnp.int32),
            pltpu.VMEM((C, HID), jnp.float32),
            pltpu.VMEM_SHARED((ACCR, HID), jnp.float32),
            pltpu.SemaphoreType.DMA,
            pltpu.SemaphoreType.DMA,
        ],
    )(_deg_sc)
    edge = functools.partial(
        pl.kernel,
        mesh=sc_mesh,
        out_type=jax.ShapeDtypeStruct((NC, HALFN, HID), jnp.float32),
        scratch_types=[
            pltpu.VMEM((NM2, C), jnp.int32),
            pltpu.VMEM((NM2, C), jnp.int32),
            pltpu.VMEM((REM2,), jnp.int32),
            pltpu.VMEM((REM2,), jnp.int32),
            pltpu.VMEM((C, HID), jnp.float32),
            pltpu.VMEM((C, HID), jnp.float32),
            pltpu.VMEM((REM2, HID), jnp.float32),
            pltpu.VMEM_SHARED((ACCR, HID), jnp.float32),
            pltpu.SemaphoreType.DMA,
            pltpu.SemaphoreType.DMA,
            pltpu.SemaphoreType.DMA,
            pltpu.SemaphoreType.DMA,
        ],
    )(_edge_sc)
    return deg, edge


def _deg_sc(idx_m_hbm, idx_r_hbm, ones_hbm, zeros_hbm, out_hbm,
            idx_v, idxr_v, ones_v, acc_sh, s0, s1):
    c = lax.axis_index("c")
    s = lax.axis_index("s")
    pltpu.sync_copy(ones_hbm, ones_v)
    for k in range(2):
        pltpu.sync_copy(idx_m_hbm.at[k, c, s], idx_v)
        pltpu.sync_copy(idx_r_hbm.at[k, c, s], idxr_v)
        pltpu.sync_copy(zeros_hbm, acc_sh.at[pl.ds(s * RPS2, RPS2)])
        plsc.subcore_barrier()

        def sc_desc(sem, i):
            return pltpu.make_async_copy(ones_v, acc_sh.at[idx_v.at[i]], sem)

        sc_desc(s0, 0).start(add=True)
        sc_desc(s1, 1).start(add=True)

        def body(r, carry):
            i = 2 * r + 2
            sc_desc(s0, i - 2).wait()
            sc_desc(s0, i).start(add=True)
            sc_desc(s1, i - 1).wait()
            sc_desc(s1, i + 1).start(add=True)
            return carry

        lax.fori_loop(0, (NM2 - 2) // 2, body, 0)
        sc_desc(s0, NM2 - 2).wait()
        sc_desc(s1, NM2 - 1).wait()
        pltpu.sync_copy(ones_v.at[pl.ds(0, REM2)], acc_sh.at[idxr_v],
                        add=True)
        plsc.subcore_barrier()
        pltpu.sync_copy(acc_sh.at[pl.ds(s * RPS2, RPS2)],
                        out_hbm.at[c, k, pl.ds(s * RPS2, RPS2)])


def _edge_sc(src_m_hbm, dst2_m_hbm, src_r_hbm, dst2_r_hbm, zeros_hbm, hs_hbm,
             out_hbm, src_v, dst_v, srcr_v, dstr_v, rows0, rows1,
             rowsr, acc_sh, g0, g1, s0, s1):
    c = lax.axis_index("c")
    s = lax.axis_index("s")
    rows = (rows0, rows1)
    gsem = (g0, g1)
    ssem = (s0, s1)
    pltpu.sync_copy(src_m_hbm.at[s], src_v)
    pltpu.sync_copy(dst2_m_hbm.at[c, s], dst_v)
    pltpu.sync_copy(src_r_hbm.at[s], srcr_v)
    pltpu.sync_copy(dst2_r_hbm.at[c, s], dstr_v)
    pltpu.sync_copy(zeros_hbm, acc_sh.at[pl.ds(s * RPS2, RPS2)])
    plsc.subcore_barrier()

    def gather(b, i):
        return pltpu.make_async_copy(hs_hbm.at[src_v.at[i]], rows[b], gsem[b])

    def scatter(b, i):
        return pltpu.make_async_copy(rows[b], acc_sh.at[dst_v.at[i]], ssem[b])

    for b in range(NB):
        gather(b, b).start()

    def round_body(r, carry):
        for b in range(NB):
            i = r * NB + b
            gather(b, i).wait()
            scatter(b, i).start(add=True)
            scatter(b, i).wait()

            @pl.when(r < NR2 - 1)
            def _():
                gather(b, i + NB).start()

        return carry

    lax.fori_loop(0, NR2, round_body, 0)
    pltpu.sync_copy(hs_hbm.at[srcr_v], rowsr)
    pltpu.sync_copy(rowsr, acc_sh.at[dstr_v], add=True)
    plsc.subcore_barrier()
    pltpu.sync_copy(acc_sh.at[pl.ds(s * RPS2, RPS2)],
                    out_hbm.at[c, pl.ds(s * RPS2, RPS2)])


def _embed_body(h_ref, w_ref, b_ref, out_ref):
    out_ref[...] = (
        jnp.dot(h_ref[...], w_ref[...], preferred_element_type=jnp.float32)
        + b_ref[...]
    )


def _idxprep_body(ei_ref, out_ref):
    for k in range(2):
        d = ei_ref[k]
        junk = HALFN + (d & (JR - 1))
        out_ref[k, 0] = jnp.where(d < HALFN, d, junk)
        out_ref[k, 1] = jnp.where(d >= HALFN, d - HALFN, junk)


def _prep_body(h1_ref, deg_ref, a_ref, b_ref, hs_ref):
    p = deg_ref[...]
    deg_o = jnp.concatenate([p[0, 0], p[1, 0]], axis=0)[:N, 0:1]
    deg_i = jnp.concatenate([p[0, 1], p[1, 1]], axis=0)[:N, 0:1]
    a = lax.rsqrt(jnp.maximum(deg_o, 1.0))
    b = lax.rsqrt(jnp.maximum(deg_i, 1.0))
    a_ref[...] = a
    b_ref[...] = b
    hs_ref[...] = h1_ref[...] * a


def _layer_body(p_ref, h_ref, a_ref, b_ref, w_ref, bias_ref, g_ref, beta_ref,
                hout_ref, hsout_ref):
    p = p_ref[...]
    agg = jnp.concatenate([p[0], p[1]], axis=0)[:N] * b_ref[...]
    hn = (
        jnp.dot(agg, w_ref[...], preferred_element_type=jnp.float32)
        + bias_ref[...]
    )
    mu = jnp.mean(hn, axis=0, keepdims=True)
    xc = hn - mu
    var = jnp.mean(xc * xc, axis=0, keepdims=True)
    hn = g_ref[...] * xc * lax.rsqrt(var + 1e-5) + beta_ref[...]
    hnew = h_ref[...] + jnp.maximum(hn, 0.0)
    hout_ref[...] = hnew
    hsout_ref[...] = hnew * a_ref[...]


def _head_body(h_ref, w0_ref, b0_ref, w1_ref, b1_ref, w2_ref, b2_ref, out_ref):
    hg = jnp.mean(h_ref[...], axis=0, keepdims=True)
    y = jnp.dot(hg, w0_ref[...], preferred_element_type=jnp.float32) + b0_ref[...]
    y = jnp.maximum(y, 0.0)
    y = jnp.dot(y, w1_ref[...], preferred_element_type=jnp.float32) + b1_ref[...]
    y = jnp.maximum(y, 0.0)
    out_ref[...] = (
        jnp.dot(y, w2_ref[...], preferred_element_type=jnp.float32) + b2_ref[...]
    )


_embed = pl.pallas_call(
    _embed_body, out_shape=jax.ShapeDtypeStruct((N, HID), jnp.float32)
)
_idxprep = pl.pallas_call(
    _idxprep_body,
    out_shape=jax.ShapeDtypeStruct((2, 2, E // C, C), jnp.int32),
)
_prep = pl.pallas_call(
    _prep_body,
    out_shape=(
        jax.ShapeDtypeStruct((N, 1), jnp.float32),
        jax.ShapeDtypeStruct((N, 1), jnp.float32),
        jax.ShapeDtypeStruct((N, HID), jnp.float32),
    ),
)
_layer = pl.pallas_call(
    _layer_body,
    out_shape=(
        jax.ShapeDtypeStruct((N, HID), jnp.float32),
        jax.ShapeDtypeStruct((N, HID), jnp.float32),
    ),
)
_head = pl.pallas_call(
    _head_body, out_shape=jax.ShapeDtypeStruct((1, 10), jnp.float32)
)


def kernel(h, e, edge_index, W_emb_h, b_emb_h, W_emb_e, b_emb_e, W_conv,
           b_conv, gamma, beta, W_mlp0, b_mlp0, W_mlp1, b_mlp1, W_mlp2,
           b_mlp2):
    src = edge_index[0]
    src_s = src.reshape(NS, EPS)
    src_m2 = src_s[:, : NM2 * C].reshape(NS, NM2, C)
    src_r2 = src_s[:, NM2 * C:]
    ei2 = _idxprep(edge_index.reshape(2, E // C, C)).reshape(2, NC, NS, EPS)
    idx_m = ei2[:, :, :, : NM2 * C].reshape(2, NC, NS, NM2, C)
    idx_r = ei2[:, :, :, NM2 * C:]
    ones_acc = jnp.ones((C, HID), jnp.float32)
    zeros_acc = jnp.zeros((RPS2, HID), jnp.float32)

    deg_sc, edge_sc = _sc_kernels()
    h1 = _embed(h, W_emb_h, b_emb_h.reshape(1, HID))
    degs = deg_sc(idx_m, idx_r, ones_acc, zeros_acc)
    a, b, hs = _prep(h1, degs)
    n_layers = W_conv.shape[0]
    for l in range(n_layers):
        p = edge_sc(src_m2, idx_m[1], src_r2, idx_r[1], zeros_acc, hs)
        h1, hs = _layer(
            p, h1, a, b, W_conv[l], b_conv[l].reshape(1, HID),
            gamma[l].reshape(1, HID), beta[l].reshape(1, HID),
        )
    return _head(
        h1, W_mlp0, b_mlp0.reshape(1, -1), W_mlp1, b_mlp1.reshape(1, -1),
        W_mlp2, b_mlp2.reshape(1, -1),
    )


# speedup vs baseline: 11.9912x; 1.2265x over previous
import functools

import jax
import jax.numpy as jnp
from jax import lax
from jax.experimental import pallas as pl
from jax.experimental.pallas import tpu as pltpu
from jax.experimental.pallas import tpu_sc as plsc

N = 10000
E = 320000
HID = 128

NC = 2
NS = 16
C = 128
NB = 2
NPAD = 10240

NW = NC * NS
EPW = E // NW
DM = EPW // C
DREM = EPW - DM * C

EPS = E // NS
NM2 = EPS // C
REM2 = EPS - NM2 * C
NR2 = NM2 // NB
HALFN = NPAD // 2
JR = 512
ACCR = HALFN + JR
RPS2 = HALFN // NS


@functools.cache
def _sc_kernels():
    sc_mesh = plsc.VectorSubcoreMesh(
        core_axis_name="c", subcore_axis_name="s", num_cores=NC,
        num_subcores=NS,
    )
    deg = functools.partial(
        pl.kernel,
        mesh=sc_mesh,
        compiler_params=pltpu.CompilerParams(needs_layout_passes=False),
        out_type=jax.ShapeDtypeStruct((NW, 2, NPAD), jnp.float32),
        scratch_types=[
            pltpu.VMEM((DM, C), jnp.int32),
            pltpu.VMEM((DM, C), jnp.int32),
            pltpu.VMEM((DREM,), jnp.int32),
            pltpu.VMEM((DREM,), jnp.int32),
            pltpu.VMEM((NPAD,), jnp.float32),
            pltpu.VMEM((NPAD,), jnp.float32),
        ],
    )(_deg_sc)
    edge = functools.partial(
        pl.kernel,
        mesh=sc_mesh,
        out_type=jax.ShapeDtypeStruct((NC, HALFN, HID), jnp.float32),
        scratch_types=[
            pltpu.VMEM((NM2, C), jnp.int32),
            pltpu.VMEM((NM2, C), jnp.int32),
            pltpu.VMEM((REM2,), jnp.int32),
            pltpu.VMEM((REM2,), jnp.int32),
            pltpu.VMEM((C, HID), jnp.float32),
            pltpu.VMEM((C, HID), jnp.float32),
            pltpu.VMEM((REM2, HID), jnp.float32),
            pltpu.VMEM_SHARED((ACCR, HID), jnp.float32),
            pltpu.SemaphoreType.DMA,
            pltpu.SemaphoreType.DMA,
            pltpu.SemaphoreType.DMA,
            pltpu.SemaphoreType.DMA,
        ],
    )(_edge_sc)
    return deg, edge


def _deg_sc(src_m_hbm, dst_m_hbm, src_r_hbm, dst_r_hbm, out_hbm,
            src_v, dst_v, srcr_v, dstr_v, hist_o, hist_i):
    c = lax.axis_index("c")
    s = lax.axis_index("s")
    w = c * NS + s
    pltpu.sync_copy(src_m_hbm.at[w], src_v)
    pltpu.sync_copy(dst_m_hbm.at[w], dst_v)
    pltpu.sync_copy(src_r_hbm.at[w], srcr_v)
    pltpu.sync_copy(dst_r_hbm.at[w], dstr_v)
    zeros16 = jnp.zeros((16,), jnp.float32)

    def zbody(i, carry):
        hist_o[pl.ds(i * 16, 16)] = zeros16
        hist_i[pl.ds(i * 16, 16)] = zeros16
        return carry

    lax.fori_loop(0, NPAD // 16, zbody, 0)
    ones16 = jnp.ones((16,), jnp.float32)

    def body(i, carry):
        for k in range(C // 16):
            plsc.addupdate_scatter(hist_o, [src_v[i, pl.ds(k * 16, 16)]],
                                   ones16)
            plsc.addupdate_scatter(hist_i, [dst_v[i, pl.ds(k * 16, 16)]],
                                   ones16)
        return carry

    lax.fori_loop(0, DM, body, 0)
    plsc.addupdate_scatter(hist_o, [srcr_v[...]], ones16)
    plsc.addupdate_scatter(hist_i, [dstr_v[...]], ones16)
    pltpu.sync_copy(hist_o, out_hbm.at[w, 0])
    pltpu.sync_copy(hist_i, out_hbm.at[w, 1])


def _edge_sc(src_m_hbm, dst2_m_hbm, src_r_hbm, dst2_r_hbm, zeros_hbm, hs_hbm,
             out_hbm, src_v, dst_v, srcr_v, dstr_v, rows0, rows1,
             rowsr, acc_sh, g0, g1, s0, s1):
    c = lax.axis_index("c")
    s = lax.axis_index("s")
    rows = (rows0, rows1)
    gsem = (g0, g1)
    ssem = (s0, s1)
    pltpu.sync_copy(src_m_hbm.at[s], src_v)
    pltpu.sync_copy(dst2_m_hbm.at[c, s], dst_v)
    pltpu.sync_copy(src_r_hbm.at[s], srcr_v)
    pltpu.sync_copy(dst2_r_hbm.at[c, s], dstr_v)
    pltpu.sync_copy(zeros_hbm, acc_sh.at[pl.ds(s * RPS2, RPS2)])
    plsc.subcore_barrier()

    def gather(b, i):
        return pltpu.make_async_copy(hs_hbm.at[src_v.at[i]], rows[b], gsem[b])

    def scatter(b, i):
        return pltpu.make_async_copy(rows[b], acc_sh.at[dst_v.at[i]], ssem[b])

    for b in range(NB):
        gather(b, b).start()

    def round_body(r, carry):
        for b in range(NB):
            i = r * NB + b
            gather(b, i).wait()
            scatter(b, i).start(add=True)
            scatter(b, i).wait()

            @pl.when(r < NR2 - 1)
            def _():
                gather(b, i + NB).start()

        return carry

    lax.fori_loop(0, NR2, round_body, 0)
    pltpu.sync_copy(hs_hbm.at[srcr_v], rowsr)
    pltpu.sync_copy(rowsr, acc_sh.at[dstr_v], add=True)
    plsc.subcore_barrier()
    pltpu.sync_copy(acc_sh.at[pl.ds(s * RPS2, RPS2)],
                    out_hbm.at[c, pl.ds(s * RPS2, RPS2)])


def _embed_body(h_ref, w_ref, b_ref, out_ref):
    out_ref[...] = (
        jnp.dot(h_ref[...], w_ref[...], preferred_element_type=jnp.float32)
        + b_ref[...]
    )


def _idxprep_body(ei_ref, out_ref):
    for k in range(2):
        d = ei_ref[k]
        junk = HALFN + (d & (JR - 1))
        out_ref[k, 0] = jnp.where(d < HALFN, d, junk)
        out_ref[k, 1] = jnp.where(d >= HALFN, d - HALFN, junk)


def _prep_body(h1_ref, deg_ref, a_ref, b_ref, hs_ref):
    p = deg_ref[...]
    sums = jnp.sum(p, axis=0)
    deg_o = jnp.reshape(sums[0], (NPAD, 1))[:N]
    deg_i = jnp.reshape(sums[1], (NPAD, 1))[:N]
    a = lax.rsqrt(jnp.maximum(deg_o, 1.0))
    b = lax.rsqrt(jnp.maximum(deg_i, 1.0))
    a_ref[...] = a
    b_ref[...] = b
    hs_ref[...] = h1_ref[...] * a


def _layer_body(p_ref, h_ref, a_ref, b_ref, w_ref, bias_ref, g_ref, beta_ref,
                hout_ref, hsout_ref):
    p = p_ref[...]
    agg = jnp.concatenate([p[0], p[1]], axis=0)[:N] * b_ref[...]
    hn = (
        jnp.dot(agg, w_ref[...], preferred_element_type=jnp.float32)
        + bias_ref[...]
    )
    mu = jnp.mean(hn, axis=0, keepdims=True)
    xc = hn - mu
    var = jnp.mean(xc * xc, axis=0, keepdims=True)
    hn = g_ref[...] * xc * lax.rsqrt(var + 1e-5) + beta_ref[...]
    hnew = h_ref[...] + jnp.maximum(hn, 0.0)
    hout_ref[...] = hnew
    hsout_ref[...] = hnew * a_ref[...]


def _head_body(h_ref, w0_ref, b0_ref, w1_ref, b1_ref, w2_ref, b2_ref, out_ref):
    hg = jnp.mean(h_ref[...], axis=0, keepdims=True)
    y = jnp.dot(hg, w0_ref[...], preferred_element_type=jnp.float32) + b0_ref[...]
    y = jnp.maximum(y, 0.0)
    y = jnp.dot(y, w1_ref[...], preferred_element_type=jnp.float32) + b1_ref[...]
    y = jnp.maximum(y, 0.0)
    out_ref[...] = (
        jnp.dot(y, w2_ref[...], preferred_element_type=jnp.float32) + b2_ref[...]
    )


_embed = pl.pallas_call(
    _embed_body, out_shape=jax.ShapeDtypeStruct((N, HID), jnp.float32)
)
_idxprep = pl.pallas_call(
    _idxprep_body,
    out_shape=jax.ShapeDtypeStruct((2, 2, E // C, C), jnp.int32),
)
_prep = pl.pallas_call(
    _prep_body,
    out_shape=(
        jax.ShapeDtypeStruct((N, 1), jnp.float32),
        jax.ShapeDtypeStruct((N, 1), jnp.float32),
        jax.ShapeDtypeStruct((N, HID), jnp.float32),
    ),
)
_layer = pl.pallas_call(
    _layer_body,
    out_shape=(
        jax.ShapeDtypeStruct((N, HID), jnp.float32),
        jax.ShapeDtypeStruct((N, HID), jnp.float32),
    ),
)
_head = pl.pallas_call(
    _head_body, out_shape=jax.ShapeDtypeStruct((1, 10), jnp.float32)
)


def kernel(h, e, edge_index, W_emb_h, b_emb_h, W_emb_e, b_emb_e, W_conv,
           b_conv, gamma, beta, W_mlp0, b_mlp0, W_mlp1, b_mlp1, W_mlp2,
           b_mlp2):
    src = edge_index[0]
    dst = edge_index[1]
    src_w = src.reshape(NW, EPW)
    dst_w = dst.reshape(NW, EPW)
    src_dm = src_w[:, : DM * C].reshape(NW, DM, C)
    dst_dm = dst_w[:, : DM * C].reshape(NW, DM, C)
    src_dr = src_w[:, DM * C:]
    dst_dr = dst_w[:, DM * C:]
    src_s = src.reshape(NS, EPS)
    src_m2 = src_s[:, : NM2 * C].reshape(NS, NM2, C)
    src_r2 = src_s[:, NM2 * C:]
    ei2 = _idxprep(edge_index.reshape(2, E // C, C)).reshape(2, NC, NS, EPS)
    idx_m = ei2[:, :, :, : NM2 * C].reshape(2, NC, NS, NM2, C)
    idx_r = ei2[:, :, :, NM2 * C:]
    zeros_acc = jnp.zeros((RPS2, HID), jnp.float32)

    deg_sc, edge_sc = _sc_kernels()
    h1 = _embed(h, W_emb_h, b_emb_h.reshape(1, HID))
    degs = deg_sc(src_dm, dst_dm, src_dr, dst_dr)
    a, b, hs = _prep(h1, degs)
    n_layers = W_conv.shape[0]
    for l in range(n_layers):
        p = edge_sc(src_m2, idx_m[1], src_r2, idx_r[1], zeros_acc, hs)
        h1, hs = _layer(
            p, h1, a, b, W_conv[l], b_conv[l].reshape(1, HID),
            gamma[l].reshape(1, HID), beta[l].reshape(1, HID),
        )
    return _head(
        h1, W_mlp0, b_mlp0.reshape(1, -1), W_mlp1, b_mlp1.reshape(1, -1),
        W_mlp2, b_mlp2.reshape(1, -1),
    )
